# baseline (speedup 1.0000x reference)
import jax
import jax.numpy as jnp
from jax.experimental import pallas as pl
from jax.experimental.pallas import tpu as pltpu

N = 4096
F = 64
BM = 512
BK = 512


def _fused_kernel(b_ref, adj_ref, w_ref, bias_ref, out_ref, sup_ref):
    k = pl.program_id(1)

    @pl.when((pl.program_id(0) == 0) & (k == 0))
    def _():
        sup_ref[...] = jnp.dot(
            b_ref[...], w_ref[...], preferred_element_type=jnp.float32
        ).astype(jnp.bfloat16)

    adj_bf = adj_ref[...].astype(jnp.bfloat16)
    sup_blk = sup_ref[pl.ds(k * BK, BK), :]
    part = jnp.dot(adj_bf, sup_blk, preferred_element_type=jnp.float32)

    @pl.when(k == 0)
    def _():
        out_ref[...] = part + bias_ref[...]

    @pl.when(k != 0)
    def _():
        out_ref[...] += part


def kernel(b_input, adj, a_weight, a_bias):
    bias2d = a_bias.reshape(1, F)
    grid = (N // BM, N // BK)
    return pl.pallas_call(
        _fused_kernel,
        grid=grid,
        in_specs=[
            pl.BlockSpec((N, F), lambda i, k: (0, 0)),
            pl.BlockSpec((BM, BK), lambda i, k: (i, k)),
            pl.BlockSpec((F, F), lambda i, k: (0, 0)),
            pl.BlockSpec((1, F), lambda i, k: (0, 0)),
        ],
        out_specs=pl.BlockSpec((BM, F), lambda i, k: (i, 0)),
        out_shape=jax.ShapeDtypeStruct((N, F), jnp.float32),
        scratch_shapes=[pltpu.VMEM((N, F), jnp.bfloat16)],
    )(b_input, adj, a_weight, bias2d)


# speedup vs baseline: 1.8419x; 1.8419x over previous
import jax
import jax.numpy as jnp
from jax.experimental import pallas as pl
from jax.experimental.pallas import tpu as pltpu

N = 4096
F = 64
BM = 256


def _fused_kernel(b_ref, adj_ref, w_ref, bias_ref, out_ref, sup_ref):
    @pl.when(pl.program_id(0) == 0)
    def _():
        sup_ref[...] = jnp.dot(
            b_ref[...], w_ref[...], preferred_element_type=jnp.float32
        ).astype(jnp.bfloat16)

    adj_bf = adj_ref[...].astype(jnp.bfloat16)
    out_ref[...] = (
        jnp.dot(adj_bf, sup_ref[...], preferred_element_type=jnp.float32)
        + bias_ref[...]
    )


def kernel(b_input, adj, a_weight, a_bias):
    bias2d = a_bias.reshape(1, F)
    grid = (N // BM,)
    return pl.pallas_call(
        _fused_kernel,
        grid=grid,
        in_specs=[
            pl.BlockSpec((N, F), lambda i: (0, 0)),
            pl.BlockSpec((BM, N), lambda i: (i, 0)),
            pl.BlockSpec((F, F), lambda i: (0, 0)),
            pl.BlockSpec((1, F), lambda i: (0, 0)),
        ],
        out_specs=pl.BlockSpec((BM, F), lambda i: (i, 0)),
        out_shape=jax.ShapeDtypeStruct((N, F), jnp.float32),
        scratch_shapes=[pltpu.VMEM((N, F), jnp.bfloat16)],
    )(b_input, adj, a_weight, bias2d)


# speedup vs baseline: 2.0017x; 1.0868x over previous
import jax
import jax.numpy as jnp
from jax.experimental import pallas as pl
from jax.experimental.pallas import tpu as pltpu

N = 4096
F = 64
BM = 256
NSTEPS = N // BM
NBUF = 4


def _fused_kernel(b_ref, adj_hbm, w_ref, bias_ref, out_ref, buf_ref, sup_ref,
                  sem_ref):
    i = pl.program_id(0)

    def _copy(block, slot):
        return pltpu.make_async_copy(
            adj_hbm.at[pl.ds(block * BM, BM), :],
            buf_ref.at[slot],
            sem_ref.at[slot],
        )

    @pl.when(i == 0)
    def _():
        for j in range(NBUF):
            _copy(j, j).start()
        sup_ref[...] = jnp.dot(
            b_ref[...], w_ref[...], preferred_element_type=jnp.float32
        ).astype(jnp.bfloat16)

    slot = jax.lax.rem(i, NBUF)
    _copy(i, slot).wait()
    adj_bf = buf_ref[slot].astype(jnp.bfloat16)
    out_ref[...] = (
        jnp.dot(adj_bf, sup_ref[...], preferred_element_type=jnp.float32)
        + bias_ref[...]
    )

    nxt = i + NBUF

    @pl.when(nxt < NSTEPS)
    def _():
        _copy(nxt, slot).start()


def kernel(b_input, adj, a_weight, a_bias):
    bias2d = a_bias.reshape(1, F)
    return pl.pallas_call(
        _fused_kernel,
        grid=(NSTEPS,),
        in_specs=[
            pl.BlockSpec((N, F), lambda i: (0, 0)),
            pl.BlockSpec(memory_space=pltpu.MemorySpace.HBM),
            pl.BlockSpec((F, F), lambda i: (0, 0)),
            pl.BlockSpec((1, F), lambda i: (0, 0)),
        ],
        out_specs=pl.BlockSpec((BM, F), lambda i: (i, 0)),
        out_shape=jax.ShapeDtypeStruct((N, F), jnp.float32),
        scratch_shapes=[
            pltpu.VMEM((NBUF, BM, N), jnp.float32),
            pltpu.VMEM((N, F), jnp.bfloat16),
            pltpu.SemaphoreType.DMA((NBUF,)),
        ],
    )(b_input, adj, a_weight, bias2d)
